# baseline (speedup 1.0000x reference)
import jax
import jax.numpy as jnp
from jax import lax
from jax.experimental import pallas as pl
from jax.experimental.pallas import tpu as pltpu
from jax.experimental.pallas import tpu_sc as plsc

NC = 2
NS = 16
NW = NC * NS


def _gather_body(table_hbm, x_hbm, out_hbm, idx_v, rows_v, gsems):
    n_b = x_hbm.shape[0] // NW
    n_l = x_hbm.shape[1]
    wid = lax.axis_index("s") * NC + lax.axis_index("c")
    b0 = wid * n_b

    pltpu.sync_copy(x_hbm.at[pl.ds(b0, n_b)], idx_v)

    pltpu.async_copy(table_hbm.at[idx_v.at[0]], rows_v.at[0], gsems.at[0])

    def step(r, carry):
        buf = lax.rem(r, 2)
        nbuf = lax.rem(r + 1, 2)

        @pl.when(r + 1 < n_b)
        def _():
            pltpu.async_copy(
                table_hbm.at[idx_v.at[r + 1]], rows_v.at[nbuf], gsems.at[nbuf]
            )

        pltpu.make_async_copy(
            table_hbm.at[idx_v.at[r]], rows_v.at[buf], gsems.at[buf]
        ).wait()
        pltpu.sync_copy(rows_v.at[buf], out_hbm.at[b0 + r])
        return carry

    lax.fori_loop(0, n_b, step, 0)


def kernel(x, table):
    b, l = x.shape
    emb = table.shape[1]
    assert b % NW == 0
    n_b = b // NW

    mesh = plsc.VectorSubcoreMesh(
        core_axis_name="c", subcore_axis_name="s", num_cores=NC, num_subcores=NS
    )
    run = pl.kernel(
        _gather_body,
        out_type=jax.ShapeDtypeStruct((b, l, emb), table.dtype),
        mesh=mesh,
        scratch_types=[
            pltpu.VMEM((n_b, l), jnp.int32),
            pltpu.VMEM((2, l, emb), jnp.float32),
            pltpu.SemaphoreType.DMA((2,)),
        ],
        compiler_params=pltpu.CompilerParams(use_tc_tiling_on_sc=False),
    )
    return run(table, x.astype(jnp.int32))


# speedup vs baseline: 1.0869x; 1.0869x over previous
import jax
import jax.numpy as jnp
from jax import lax
from jax.experimental import pallas as pl
from jax.experimental.pallas import tpu as pltpu
from jax.experimental.pallas import tpu_sc as plsc

NC = 2
NS = 16
NW = NC * NS
LANES = 128
NBUF = 4
VL = 16


def _gather_body(n_l, table_hbm, xp_hbm, out_hbm, idx_v, idx_c, rows_v, gsems):
    n_b = xp_hbm.shape[0] // NW
    n_l8 = idx_v.shape[1]
    wid = lax.axis_index("s") * NC + lax.axis_index("c")
    b0 = wid * n_b

    pltpu.sync_copy(xp_hbm.at[pl.ds(b0, n_b), pl.ds(0, n_l8)], idx_v)

    ks = list(range(0, n_l - VL, VL)) + [n_l - VL]

    def prep_gather(r, buf):
        for k in ks:
            idx_c[r, pl.ds(k, VL)] = idx_v[r, pl.ds(k, VL)]
        pltpu.async_copy(
            table_hbm.at[idx_c.at[r]], rows_v.at[buf], gsems.at[buf]
        )

    for r in range(NBUF - 1):
        prep_gather(r, r)

    def step(r, carry):
        buf = lax.rem(r, NBUF)

        @pl.when(r + NBUF - 1 < n_b)
        def _():
            prep_gather(r + NBUF - 1, lax.rem(r + NBUF - 1, NBUF))

        pltpu.make_async_copy(
            table_hbm.at[idx_c.at[r]], rows_v.at[buf], gsems.at[buf]
        ).wait()
        pltpu.sync_copy(rows_v.at[buf], out_hbm.at[b0 + r])
        return carry

    lax.fori_loop(0, n_b, step, 0)


def kernel(x, table):
    b, l = x.shape
    emb = table.shape[1]
    assert b % NW == 0
    l8 = (l + 7) // 8 * 8

    xp = jnp.pad(x.astype(jnp.int32), ((0, 0), (0, LANES - l)))

    mesh = plsc.VectorSubcoreMesh(
        core_axis_name="c", subcore_axis_name="s", num_cores=NC, num_subcores=NS
    )
    run = pl.kernel(
        lambda *args: _gather_body(l, *args),
        out_type=jax.ShapeDtypeStruct((b, l, emb), table.dtype),
        mesh=mesh,
        scratch_types=[
            pltpu.VMEM((b // NW, l8), jnp.int32),
            pltpu.VMEM((b // NW, l), jnp.int32),
            pltpu.VMEM((NBUF, l, emb), jnp.float32),
            pltpu.SemaphoreType.DMA((NBUF,)),
        ],
        compiler_params=pltpu.CompilerParams(use_tc_tiling_on_sc=False),
    )
    return run(table, xp)


# speedup vs baseline: 1.1071x; 1.0185x over previous
import jax
import jax.numpy as jnp
from jax import lax
from jax.experimental import pallas as pl
from jax.experimental.pallas import tpu as pltpu
from jax.experimental.pallas import tpu_sc as plsc

NC = 2
NS = 16
NW = NC * NS
LANES = 128
CH = 128
NBUF = 4
VL = 16


def _mesh():
    return plsc.VectorSubcoreMesh(
        core_axis_name="c", subcore_axis_name="s", num_cores=NC, num_subcores=NS
    )


def _wid():
    return lax.axis_index("s") * NC + lax.axis_index("c")


def _repack_body(n_l, xp_hbm, idx3_hbm, slab_v, flat_v, sem):
    n_b = xp_hbm.shape[0] // NW
    n_chunks = idx3_hbm.shape[1]
    wid = _wid()

    pltpu.sync_copy(xp_hbm.at[pl.ds(wid * n_b, n_b)], slab_v)

    ks = list(range(0, n_l - VL, VL)) + [n_l - VL]

    def rloop(r, carry):
        for k in ks:
            flat_v[pl.ds(n_l * r + k, VL)] = slab_v[r, pl.ds(k, VL)]
        return carry

    lax.fori_loop(0, n_b, rloop, 0)

    def wstart(rr, carry):
        pltpu.async_copy(
            flat_v.at[pl.ds(LANES * rr, LANES)], idx3_hbm.at[wid, rr], sem
        )
        return carry

    lax.fori_loop(0, n_chunks, wstart, 0)

    def wdrain(rr, carry):
        pltpu.make_async_copy(
            flat_v.at[pl.ds(LANES * rr, LANES)], idx3_hbm.at[wid, rr], sem
        ).wait()
        return carry

    lax.fori_loop(0, n_chunks, wdrain, 0)


def _gather_body(table_hbm, idx3_hbm, out_hbm, idx_v, rows_v, gsems):
    n_chunks = idx3_hbm.shape[1]
    wid = _wid()
    base = wid * (n_chunks * CH)

    pltpu.sync_copy(idx3_hbm.at[wid], idx_v)

    def start_gather(j, buf):
        pltpu.async_copy(
            table_hbm.at[idx_v.at[j]], rows_v.at[buf], gsems.at[buf]
        )

    for j in range(NBUF - 1):
        start_gather(j, j)

    def step(j, carry):
        buf = lax.rem(j, NBUF)

        @pl.when(j + NBUF - 1 < n_chunks)
        def _():
            start_gather(j + NBUF - 1, lax.rem(j + NBUF - 1, NBUF))

        pltpu.make_async_copy(
            table_hbm.at[idx_v.at[j]], rows_v.at[buf], gsems.at[buf]
        ).wait()
        pltpu.sync_copy(rows_v.at[buf], out_hbm.at[pl.ds(base + j * CH, CH)])
        return carry

    lax.fori_loop(0, n_chunks, step, 0)


def kernel(x, table):
    b, l = x.shape
    emb = table.shape[1]
    total = b * l
    assert total % (NW * CH) == 0
    n_chunks = total // (NW * CH)
    n_b = b // NW

    xp = jnp.pad(x.astype(jnp.int32), ((0, 0), (0, LANES - l)))

    repack = pl.kernel(
        lambda *args: _repack_body(l, *args),
        out_type=jax.ShapeDtypeStruct((NW, n_chunks, CH), jnp.int32),
        mesh=_mesh(),
        scratch_types=[
            pltpu.VMEM((n_b, LANES), jnp.int32),
            pltpu.VMEM((n_b * l,), jnp.int32),
            pltpu.SemaphoreType.DMA,
        ],
        compiler_params=pltpu.CompilerParams(use_tc_tiling_on_sc=True),
    )
    idx3 = repack(xp)

    gather = pl.kernel(
        _gather_body,
        out_type=jax.ShapeDtypeStruct((total, emb), table.dtype),
        mesh=_mesh(),
        scratch_types=[
            pltpu.VMEM((n_chunks, CH), jnp.int32),
            pltpu.VMEM((NBUF, CH, emb), jnp.float32),
            pltpu.SemaphoreType.DMA((NBUF,)),
        ],
        compiler_params=pltpu.CompilerParams(use_tc_tiling_on_sc=False),
    )
    out = gather(table, idx3)
    return out.reshape(b, l, emb)
